# Initial kernel scaffold; baseline (speedup 1.0000x reference)
import jax
import jax.numpy as jnp
from jax.experimental import pallas as pl


def kernel(p, params):
    raise NotImplementedError("write your pallas kernel here")


# speedup vs baseline: 1.3169x; 1.3169x over previous
import functools

import jax
import jax.numpy as jnp
from jax import lax
from jax.experimental import pallas as pl
from jax.experimental.pallas import tpu as pltpu
from jax.experimental.pallas import tpu_sc as plsc

EPS = 1e-6
T = 1024
KNN = 20
HID = 128
RESO = 64
NSEG = RESO * RESO
PADDING = 0.1
SCALE = 8.0
NEG = -1e30

_HIGH = lax.Precision.HIGHEST


def _mm(w, x):
    return lax.dot_general(w, x, (((1,), (0,)), ((), ())),
                           precision=lax.Precision.DEFAULT,
                           preferred_element_type=jnp.float32)


def _mm3(w, xs):
    return [_mm(w, x) for x in xs]


def _vn_lrelu3(ps, ds):
    dot = ps[0] * ds[0] + ps[1] * ds[1] + ps[2] * ds[2]
    dsq = ds[0] * ds[0] + ds[1] * ds[1] + ds[2] * ds[2]
    coef = jnp.minimum(dot, 0.0) / (dsq + EPS)
    return [p - coef * d for p, d in zip(ps, ds)]


def _resblock3(xs, wd0, wfc0, wd1, wfc1, wsc):
    ys = _vn_lrelu3(xs, _mm3(wd0, xs))
    net = _mm3(wfc0, ys)
    zs = _vn_lrelu3(net, _mm3(wd1, net))
    dxs = _mm3(wfc1, zs)
    xss = _mm3(wsc, xs)
    return [a + b for a, b in zip(xss, dxs)]


TB = 256


def _knn_body(pt_ref, ptb_ref, wf_ref, wd_ref, out_ref):
    pt = pt_ref[0]
    ptb = ptb_ref[0]
    wf = wf_ref[...]
    wd = wd_ref[...]
    g = lax.dot_general(pt, ptb, (((0,), (0,)), ((), ())),
                        precision=lax.Precision.DEFAULT,
                        preferred_element_type=jnp.float32)
    ones3 = jnp.ones((3, 1), jnp.float32)
    xx_col = lax.dot_general(pt * pt, ones3, (((0,), (0,)), ((), ())),
                             precision=_HIGH,
                             preferred_element_type=jnp.float32)
    xx_row = jnp.sum(ptb * ptb, axis=0, keepdims=True)
    cur = 2.0 * g - xx_col - xx_row
    iota_r = lax.broadcasted_iota(jnp.int32, (T, TB), 0)

    xr = [ptb[c:c + 1, :] for c in range(3)]
    accs = [jnp.zeros((HID, TB), jnp.float32) for _ in range(3)]
    for _ in range(KNN):
        m = jnp.max(cur, axis=0, keepdims=True)
        eq = cur >= m
        sel = jnp.min(jnp.where(eq, iota_r, T), axis=0, keepdims=True)
        oh = iota_r == sel
        nbr = lax.dot_general(pt, oh.astype(jnp.float32), (((1,), (0,)), ((), ())),
                              precision=_HIGH, preferred_element_type=jnp.float32)
        cur = jnp.where(oh, NEG, cur)
        nb = [nbr[c:c + 1, :] for c in range(3)]
        fx = [nb[c] - xr[c] for c in range(3)]
        cr = [nb[1] * xr[2] - nb[2] * xr[1],
              nb[2] * xr[0] - nb[0] * xr[2],
              nb[0] * xr[1] - nb[1] * xr[0]]
        ps, ds = [], []
        for c in range(3):
            feat_c = jnp.concatenate([fx[c], xr[c], cr[c]], axis=0)
            ps.append(_mm(wf, feat_c))
            ds.append(_mm(wd, feat_c))
        outs = _vn_lrelu3(ps, ds)
        for c in range(3):
            accs[c] = accs[c] + outs[c]
    for c in range(3):
        out_ref[0, c] = accs[c] * (1.0 / KNN)


def _knn_conv(pt, wf, wd):
    b = pt.shape[0]
    return pl.pallas_call(
        _knn_body,
        grid=(b, T // TB),
        in_specs=[
            pl.BlockSpec((1, 3, T), lambda i, j: (i, 0, 0)),
            pl.BlockSpec((1, 3, TB), lambda i, j: (i, 0, j)),
            pl.BlockSpec((HID, 3), lambda i, j: (0, 0)),
            pl.BlockSpec((HID, 3), lambda i, j: (0, 0)),
        ],
        out_specs=pl.BlockSpec((1, 3, HID, TB), lambda i, j: (i, 0, 0, j)),
        out_shape=jax.ShapeDtypeStruct((b, 3, HID, T), jnp.float32),
    )(pt, pt, wf, wd)


def _fcpos_block0_body(x_ref, wpos_ref, wd0_ref, wfc0_ref, wd1_ref, wfc1_ref,
                       wsc_ref, out_ref):
    xs = [x_ref[0, c] for c in range(3)]
    x1 = _mm3(wpos_ref[...], xs)
    ys = _resblock3(x1, wd0_ref[...], wfc0_ref[...], wd1_ref[...],
                    wfc1_ref[...], wsc_ref[...])
    for c in range(3):
        out_ref[0, c] = ys[c]


def _fcpos_block0(x, wpos, blk):
    b = x.shape[0]
    full = lambda s: pl.BlockSpec(s, lambda i: tuple(0 for _ in s))
    return pl.pallas_call(
        _fcpos_block0_body,
        grid=(b,),
        in_specs=[
            pl.BlockSpec((1, 3, HID, T), lambda i: (i, 0, 0, 0)),
            full((2 * HID, HID)),
            full((2 * HID, 2 * HID)),
            full((HID, 2 * HID)),
            full((HID, HID)),
            full((HID, HID)),
            full((HID, 2 * HID)),
        ],
        out_specs=pl.BlockSpec((1, 3, HID, T), lambda i: (i, 0, 0, 0)),
        out_shape=jax.ShapeDtypeStruct((b, 3, HID, T), jnp.float32),
    )(x, wpos, blk['d0'], blk['fc0'], blk['d1'], blk['fc1'], blk['sc'])


def _block_body(x_ref, p_ref, wd0_ref, wfc0_ref, wd1_ref, wfc1_ref, wsc_ref,
                *rest):
    wcc_ref, out_ref = (rest if len(rest) == 2 else (None, rest[0]))
    xs = [jnp.concatenate([x_ref[0, c], p_ref[0, c]], axis=0)
          for c in range(3)]
    ys = _resblock3(xs, wd0_ref[...], wfc0_ref[...], wd1_ref[...],
                    wfc1_ref[...], wsc_ref[...])
    if wcc_ref is not None:
        ys = _mm3(wcc_ref[...], ys)
    for c in range(3):
        out_ref[0, c] = ys[c]


def _block(x, pooled, blk, wcc=None):
    b = x.shape[0]
    full = lambda s: pl.BlockSpec(s, lambda i: tuple(0 for _ in s))
    in_specs = [
        pl.BlockSpec((1, 3, HID, T), lambda i: (i, 0, 0, 0)),
        pl.BlockSpec((1, 3, HID, T), lambda i: (i, 0, 0, 0)),
        full((2 * HID, 2 * HID)),
        full((HID, 2 * HID)),
        full((HID, HID)),
        full((HID, HID)),
        full((HID, 2 * HID)),
    ]
    args = [x, pooled, blk['d0'], blk['fc0'], blk['d1'], blk['fc1'], blk['sc']]
    if wcc is not None:
        in_specs.append(full((HID, HID)))
        args.append(wcc)
    return pl.pallas_call(
        _block_body,
        grid=(b,),
        in_specs=in_specs,
        out_specs=pl.BlockSpec((1, 3, HID, T), lambda i: (i, 0, 0, 0)),
        out_shape=jax.ShapeDtypeStruct((b, 3, HID, T), jnp.float32),
    )(*args)


_NC, _NS = 2, 16
_NW = _NC * _NS
_CB = 8


def _sc_mesh():
    return plsc.VectorSubcoreMesh(core_axis_name="c", subcore_axis_name="s",
                                  num_cores=_NC, num_subcores=_NS)


def _task_coords(tid):
    b = tid // (3 * _CB)
    r = tid % (3 * _CB)
    return b, r // _CB, r % _CB


def _pool_max(data, idx):
    b = data.shape[0]
    ntask = b * 3 * _CB

    @functools.partial(
        pl.kernel,
        out_type=jax.ShapeDtypeStruct((b, 3, HID, T), jnp.float32),
        mesh=_sc_mesh(),
        compiler_params=pltpu.CompilerParams(needs_layout_passes=False),
        scratch_types=[
            pltpu.VMEM((16, T), jnp.float32),
            pltpu.VMEM((16, NSEG), jnp.float32),
            pltpu.VMEM((16, T), jnp.float32),
            pltpu.VMEM((T,), jnp.int32),
        ],
    )
    def k(data_hbm, idx_hbm, out_hbm, data_v, fea_v, pool_v, idx_v):
        wid = lax.axis_index("s") * _NC + lax.axis_index("c")
        lanes = lax.iota(jnp.int32, 16)
        neg = jnp.full((16,), NEG, jnp.float32)

        def task(tid):
            bb, coord, cb = _task_coords(tid)
            c0 = pl.multiple_of(cb * 16, 16)
            pltpu.sync_copy(idx_hbm.at[bb], idx_v)
            pltpu.sync_copy(data_hbm.at[bb, coord, pl.ds(c0, 16), :], data_v)

            def init_body(s, carry):
                plsc.store_scatter(fea_v, [lanes, jnp.full((16,), s, jnp.int32)], neg)
                return carry
            lax.fori_loop(0, NSEG, init_body, 0)

            def max_body(tc, carry):
                iv = idx_v[pl.ds(tc * 16, 16)]
                for j in range(16):
                    col = jnp.full((16,), tc * 16 + j, jnp.int32)
                    seg = jnp.full((16,), iv[j], jnp.int32)
                    d = plsc.load_gather(data_v, [lanes, col])
                    f = plsc.load_gather(fea_v, [lanes, seg])
                    plsc.store_scatter(fea_v, [lanes, seg], jnp.maximum(f, d))
                return carry
            lax.fori_loop(0, T // 16, max_body, 0)

            def gat_body(tc, carry):
                iv = idx_v[pl.ds(tc * 16, 16)]
                for j in range(16):
                    col = jnp.full((16,), tc * 16 + j, jnp.int32)
                    seg = jnp.full((16,), iv[j], jnp.int32)
                    f = plsc.load_gather(fea_v, [lanes, seg])
                    plsc.store_scatter(pool_v, [lanes, col], f)
                return carry
            lax.fori_loop(0, T // 16, gat_body, 0)
            pltpu.sync_copy(pool_v, out_hbm.at[bb, coord, pl.ds(c0, 16), :])

        task(wid)

        @pl.when(wid + _NW < ntask)
        def _():
            task(wid + _NW)

    return k(data, idx)


def _seg_mean(data, idx):
    b = data.shape[0]
    ntask = b * 3 * _CB

    @functools.partial(
        pl.kernel,
        out_type=jax.ShapeDtypeStruct((b, 3, HID, NSEG), jnp.float32),
        mesh=_sc_mesh(),
        compiler_params=pltpu.CompilerParams(needs_layout_passes=False),
        scratch_types=[
            pltpu.VMEM((16, T), jnp.float32),
            pltpu.VMEM((16, NSEG), jnp.float32),
            pltpu.VMEM((NSEG,), jnp.float32),
            pltpu.VMEM((T,), jnp.int32),
        ],
    )
    def k(data_hbm, idx_hbm, out_hbm, data_v, sum_v, cnt_v, idx_v):
        wid = lax.axis_index("s") * _NC + lax.axis_index("c")
        lanes = lax.iota(jnp.int32, 16)
        zero = jnp.zeros((16,), jnp.float32)

        def task(tid):
            bb, coord, cb = _task_coords(tid)
            c0 = pl.multiple_of(cb * 16, 16)
            pltpu.sync_copy(idx_hbm.at[bb], idx_v)
            pltpu.sync_copy(data_hbm.at[bb, coord, pl.ds(c0, 16), :], data_v)

            def init_body(s, carry):
                plsc.store_scatter(sum_v, [lanes, jnp.full((16,), s, jnp.int32)], zero)
                return carry
            lax.fori_loop(0, NSEG, init_body, 0)

            def cnt_init_body(s, carry):
                cnt_v[pl.ds(s * 16, 16)] = zero
                return carry
            lax.fori_loop(0, NSEG // 16, cnt_init_body, 0)

            def add_body(tc, carry):
                iv = idx_v[pl.ds(tc * 16, 16)]
                for j in range(16):
                    col = jnp.full((16,), tc * 16 + j, jnp.int32)
                    seg = jnp.full((16,), iv[j], jnp.int32)
                    d = plsc.load_gather(data_v, [lanes, col])
                    plsc.addupdate_scatter(sum_v, [lanes, seg], d)
                    c = plsc.load_gather(cnt_v, [seg])
                    plsc.store_scatter(cnt_v, [seg], c + 1.0)
                return carry
            lax.fori_loop(0, T // 16, add_body, 0)

            def div_body(sc, carry):
                cvec = cnt_v[pl.ds(sc * 16, 16)]
                scl = 1.0 / jnp.maximum(cvec, 1.0)
                for j in range(16):
                    seg = jnp.full((16,), sc * 16 + j, jnp.int32)
                    v = plsc.load_gather(sum_v, [lanes, seg])
                    plsc.store_scatter(sum_v, [lanes, seg], v * scl[j])
                return carry
            lax.fori_loop(0, NSEG // 16, div_body, 0)
            pltpu.sync_copy(sum_v, out_hbm.at[bb, coord, pl.ds(c0, 16), :])

        task(wid)

        @pl.when(wid + _NW < ntask)
        def _():
            task(wid + _NW)

    return k(data, idx)


def kernel(p, params):
    b = p.shape[0]
    xy = (p / SCALE)[..., jnp.array([0, 2])]
    xy = xy / (1.0 + PADDING + 10e-4) + 0.5
    xy = jnp.where(xy >= 1.0, 1.0 - 10e-4, xy)
    xy = jnp.where(xy < 0.0, 0.0, xy)
    xyi = (xy * RESO).astype(jnp.int32)
    idx = xyi[..., 0] + RESO * xyi[..., 1]

    pt = jnp.swapaxes(p, 1, 2)
    net = _knn_conv(pt, params['conv_pos_f'], params['conv_pos_d'])
    net = _fcpos_block0(net, params['fc_pos'], params['block0'])
    for i in range(1, 5):
        pooled = _pool_max(net, idx)
        net = _block(net, pooled, params['block' + str(i)],
                     wcc=params['fc_c'] if i == 4 else None)
    mean_sc = _seg_mean(net, idx)
    return jnp.transpose(mean_sc, (0, 2, 1, 3)).reshape(b, 3 * HID, RESO, RESO)


# speedup vs baseline: 1.5582x; 1.1833x over previous
import functools

import jax
import jax.numpy as jnp
from jax import lax
from jax.experimental import pallas as pl
from jax.experimental.pallas import tpu as pltpu
from jax.experimental.pallas import tpu_sc as plsc

EPS = 1e-6
T = 1024
KNN = 20
HID = 128
RESO = 64
NSEG = RESO * RESO
PADDING = 0.1
SCALE = 8.0
NEG = -1e30

_HIGH = lax.Precision.HIGHEST


def _mm(w, x):
    return lax.dot_general(w, x, (((1,), (0,)), ((), ())),
                           precision=lax.Precision.DEFAULT,
                           preferred_element_type=jnp.float32)


def _mm3(w, xs):
    return [_mm(w, x) for x in xs]


def _vn_lrelu3(ps, ds):
    dot = ps[0] * ds[0] + ps[1] * ds[1] + ps[2] * ds[2]
    dsq = ds[0] * ds[0] + ds[1] * ds[1] + ds[2] * ds[2]
    coef = jnp.minimum(dot, 0.0) / (dsq + EPS)
    return [p - coef * d for p, d in zip(ps, ds)]


def _resblock3(xs, wd0, wfc0, wd1, wfc1, wsc):
    ys = _vn_lrelu3(xs, _mm3(wd0, xs))
    net = _mm3(wfc0, ys)
    zs = _vn_lrelu3(net, _mm3(wd1, net))
    dxs = _mm3(wfc1, zs)
    xss = _mm3(wsc, xs)
    return [a + b for a, b in zip(xss, dxs)]


TB = 256


def _knn_body(pt_ref, ptb_ref, wf_ref, wd_ref, out_ref):
    pt = pt_ref[0]
    ptb = ptb_ref[0]
    wf = wf_ref[...]
    wd = wd_ref[...]
    g = lax.dot_general(pt, ptb, (((0,), (0,)), ((), ())),
                        precision=lax.Precision.DEFAULT,
                        preferred_element_type=jnp.float32)
    ones3 = jnp.ones((3, 1), jnp.float32)
    xx_col = lax.dot_general(pt * pt, ones3, (((0,), (0,)), ((), ())),
                             precision=_HIGH,
                             preferred_element_type=jnp.float32)
    xx_row = jnp.sum(ptb * ptb, axis=0, keepdims=True)
    cur = 2.0 * g - xx_col - xx_row
    iota_r = lax.broadcasted_iota(jnp.int32, (T, TB), 0)

    xr = [ptb[c:c + 1, :] for c in range(3)]
    accs = [jnp.zeros((HID, TB), jnp.float32) for _ in range(3)]
    for _ in range(KNN):
        m = jnp.max(cur, axis=0, keepdims=True)
        eq = cur >= m
        sel = jnp.min(jnp.where(eq, iota_r, T), axis=0, keepdims=True)
        oh = iota_r == sel
        nbr = lax.dot_general(pt, oh.astype(jnp.float32), (((1,), (0,)), ((), ())),
                              precision=_HIGH, preferred_element_type=jnp.float32)
        cur = jnp.where(oh, NEG, cur)
        nb = [nbr[c:c + 1, :] for c in range(3)]
        fx = [nb[c] - xr[c] for c in range(3)]
        cr = [nb[1] * xr[2] - nb[2] * xr[1],
              nb[2] * xr[0] - nb[0] * xr[2],
              nb[0] * xr[1] - nb[1] * xr[0]]
        ps, ds = [], []
        for c in range(3):
            feat_c = jnp.concatenate([fx[c], xr[c], cr[c]], axis=0)
            ps.append(_mm(wf, feat_c))
            ds.append(_mm(wd, feat_c))
        outs = _vn_lrelu3(ps, ds)
        for c in range(3):
            accs[c] = accs[c] + outs[c]
    for c in range(3):
        out_ref[0, c] = accs[c] * (1.0 / KNN)


def _knn_conv(pt, wf, wd):
    b = pt.shape[0]
    return pl.pallas_call(
        _knn_body,
        grid=(b, T // TB),
        in_specs=[
            pl.BlockSpec((1, 3, T), lambda i, j: (i, 0, 0)),
            pl.BlockSpec((1, 3, TB), lambda i, j: (i, 0, j)),
            pl.BlockSpec((HID, 3), lambda i, j: (0, 0)),
            pl.BlockSpec((HID, 3), lambda i, j: (0, 0)),
        ],
        out_specs=pl.BlockSpec((1, 3, HID, TB), lambda i, j: (i, 0, 0, j)),
        out_shape=jax.ShapeDtypeStruct((b, 3, HID, T), jnp.float32),
    )(pt, pt, wf, wd)


def _fcpos_block0_body(x_ref, wpos_ref, wd0_ref, wfc0_ref, wd1_ref, wfc1_ref,
                       wsc_ref, out_ref):
    xs = [x_ref[0, c] for c in range(3)]
    x1 = _mm3(wpos_ref[...], xs)
    ys = _resblock3(x1, wd0_ref[...], wfc0_ref[...], wd1_ref[...],
                    wfc1_ref[...], wsc_ref[...])
    for c in range(3):
        out_ref[0, c] = ys[c]


def _fcpos_block0(x, wpos, blk):
    b = x.shape[0]
    full = lambda s: pl.BlockSpec(s, lambda i: tuple(0 for _ in s))
    return pl.pallas_call(
        _fcpos_block0_body,
        grid=(b,),
        in_specs=[
            pl.BlockSpec((1, 3, HID, T), lambda i: (i, 0, 0, 0)),
            full((2 * HID, HID)),
            full((2 * HID, 2 * HID)),
            full((HID, 2 * HID)),
            full((HID, HID)),
            full((HID, HID)),
            full((HID, 2 * HID)),
        ],
        out_specs=pl.BlockSpec((1, 3, HID, T), lambda i: (i, 0, 0, 0)),
        out_shape=jax.ShapeDtypeStruct((b, 3, HID, T), jnp.float32),
    )(x, wpos, blk['d0'], blk['fc0'], blk['d1'], blk['fc1'], blk['sc'])


def _block_body(x_ref, p_ref, wd0_ref, wfc0_ref, wd1_ref, wfc1_ref, wsc_ref,
                *rest):
    wcc_ref, out_ref = (rest if len(rest) == 2 else (None, rest[0]))
    xs = [jnp.concatenate([x_ref[0, c], p_ref[0, c]], axis=0)
          for c in range(3)]
    ys = _resblock3(xs, wd0_ref[...], wfc0_ref[...], wd1_ref[...],
                    wfc1_ref[...], wsc_ref[...])
    if wcc_ref is not None:
        ys = _mm3(wcc_ref[...], ys)
    for c in range(3):
        out_ref[0, c] = ys[c]


def _block(x, pooled, blk, wcc=None):
    b = x.shape[0]
    full = lambda s: pl.BlockSpec(s, lambda i: tuple(0 for _ in s))
    in_specs = [
        pl.BlockSpec((1, 3, HID, T), lambda i: (i, 0, 0, 0)),
        pl.BlockSpec((1, 3, HID, T), lambda i: (i, 0, 0, 0)),
        full((2 * HID, 2 * HID)),
        full((HID, 2 * HID)),
        full((HID, HID)),
        full((HID, HID)),
        full((HID, 2 * HID)),
    ]
    args = [x, pooled, blk['d0'], blk['fc0'], blk['d1'], blk['fc1'], blk['sc']]
    if wcc is not None:
        in_specs.append(full((HID, HID)))
        args.append(wcc)
    return pl.pallas_call(
        _block_body,
        grid=(b,),
        in_specs=in_specs,
        out_specs=pl.BlockSpec((1, 3, HID, T), lambda i: (i, 0, 0, 0)),
        out_shape=jax.ShapeDtypeStruct((b, 3, HID, T), jnp.float32),
    )(*args)


_NC, _NS = 2, 16
_NW = _NC * _NS
_CB = 8


def _sc_mesh():
    return plsc.VectorSubcoreMesh(core_axis_name="c", subcore_axis_name="s",
                                  num_cores=_NC, num_subcores=_NS)


def _task_coords(tid):
    b = tid // (3 * _CB)
    r = tid % (3 * _CB)
    return b, r // _CB, r % _CB


def _pool_max(data, idx):
    b = data.shape[0]
    ntask = b * 3 * _CB

    @functools.partial(
        pl.kernel,
        out_type=jax.ShapeDtypeStruct((b, 3, HID, T), jnp.float32),
        mesh=_sc_mesh(),
        compiler_params=pltpu.CompilerParams(needs_layout_passes=False),
        scratch_types=[
            pltpu.VMEM((16, T), jnp.float32),
            pltpu.VMEM((16, NSEG), jnp.float32),
            pltpu.VMEM((16, T), jnp.float32),
            pltpu.VMEM((T,), jnp.int32),
        ],
    )
    def k(data_hbm, idx_hbm, out_hbm, data_v, fea_v, pool_v, idx_v):
        wid = lax.axis_index("s") * _NC + lax.axis_index("c")
        lanes = lax.iota(jnp.int32, 16)
        neg = jnp.full((16,), NEG, jnp.float32)

        def task(tid):
            bb, coord, cb = _task_coords(tid)
            c0 = pl.multiple_of(cb * 16, 16)
            pltpu.sync_copy(idx_hbm.at[bb], idx_v)
            pltpu.sync_copy(data_hbm.at[bb, coord, pl.ds(c0, 16), :], data_v)

            def init_body(tc, carry):
                iv = idx_v[pl.ds(tc * 16, 16)]
                for j in range(16):
                    plsc.store_scatter(
                        fea_v, [lanes, jnp.full((16,), iv[j], jnp.int32)], neg)
                return carry
            lax.fori_loop(0, T // 16, init_body, 0)

            def max_body(tc, carry):
                iv = idx_v[pl.ds(tc * 16, 16)]
                for j in range(16):
                    col = jnp.full((16,), tc * 16 + j, jnp.int32)
                    seg = jnp.full((16,), iv[j], jnp.int32)
                    d = plsc.load_gather(data_v, [lanes, col])
                    f = plsc.load_gather(fea_v, [lanes, seg])
                    plsc.store_scatter(fea_v, [lanes, seg], jnp.maximum(f, d))
                return carry
            lax.fori_loop(0, T // 16, max_body, 0)

            def gat_body(tc, carry):
                iv = idx_v[pl.ds(tc * 16, 16)]
                for j in range(16):
                    col = jnp.full((16,), tc * 16 + j, jnp.int32)
                    seg = jnp.full((16,), iv[j], jnp.int32)
                    f = plsc.load_gather(fea_v, [lanes, seg])
                    plsc.store_scatter(pool_v, [lanes, col], f)
                return carry
            lax.fori_loop(0, T // 16, gat_body, 0)
            pltpu.sync_copy(pool_v, out_hbm.at[bb, coord, pl.ds(c0, 16), :])

        task(wid)

        @pl.when(wid + _NW < ntask)
        def _():
            task(wid + _NW)

    return k(data, idx)


def _seg_mean(data, idx):
    b = data.shape[0]
    ntask = b * 3 * _CB

    @functools.partial(
        pl.kernel,
        out_type=jax.ShapeDtypeStruct((b, 3, HID, NSEG), jnp.float32),
        mesh=_sc_mesh(),
        compiler_params=pltpu.CompilerParams(needs_layout_passes=False),
        scratch_types=[
            pltpu.VMEM((16, T), jnp.float32),
            pltpu.VMEM((16, NSEG), jnp.float32),
            pltpu.VMEM((NSEG,), jnp.float32),
            pltpu.VMEM((T,), jnp.int32),
        ],
    )
    def k(data_hbm, idx_hbm, out_hbm, data_v, sum_v, cnt_v, idx_v):
        wid = lax.axis_index("s") * _NC + lax.axis_index("c")
        lanes = lax.iota(jnp.int32, 16)
        zero = jnp.zeros((16,), jnp.float32)

        def task(tid):
            bb, coord, cb = _task_coords(tid)
            c0 = pl.multiple_of(cb * 16, 16)
            pltpu.sync_copy(idx_hbm.at[bb], idx_v)
            pltpu.sync_copy(data_hbm.at[bb, coord, pl.ds(c0, 16), :], data_v)

            def init_body(tc, carry):
                iv = idx_v[pl.ds(tc * 16, 16)]
                for j in range(16):
                    plsc.store_scatter(
                        sum_v, [lanes, jnp.full((16,), iv[j], jnp.int32)], zero)
                return carry
            lax.fori_loop(0, T // 16, init_body, 0)

            def cnt_init_body(s, carry):
                cnt_v[pl.ds(s * 16, 16)] = zero
                return carry
            lax.fori_loop(0, NSEG // 16, cnt_init_body, 0)

            def add_body(tc, carry):
                iv = idx_v[pl.ds(tc * 16, 16)]
                for j in range(16):
                    col = jnp.full((16,), tc * 16 + j, jnp.int32)
                    seg = jnp.full((16,), iv[j], jnp.int32)
                    d = plsc.load_gather(data_v, [lanes, col])
                    plsc.addupdate_scatter(sum_v, [lanes, seg], d)
                    c = plsc.load_gather(cnt_v, [seg])
                    plsc.store_scatter(cnt_v, [seg], c + 1.0)
                return carry
            lax.fori_loop(0, T // 16, add_body, 0)

            def div_body(sc, carry):
                cvec = cnt_v[pl.ds(sc * 16, 16)]
                scl = 1.0 / jnp.maximum(cvec, 1.0)
                for j in range(16):
                    seg = jnp.full((16,), sc * 16 + j, jnp.int32)
                    v = plsc.load_gather(sum_v, [lanes, seg])
                    occ = jnp.full((16,), cvec[j], jnp.float32) > 0.0
                    w = jnp.where(occ, v * scl[j], jnp.zeros((16,), jnp.float32))
                    plsc.store_scatter(sum_v, [lanes, seg], w)
                return carry
            lax.fori_loop(0, NSEG // 16, div_body, 0)
            pltpu.sync_copy(sum_v, out_hbm.at[bb, coord, pl.ds(c0, 16), :])

        task(wid)

        @pl.when(wid + _NW < ntask)
        def _():
            task(wid + _NW)

    return k(data, idx)


def kernel(p, params):
    b = p.shape[0]
    xy = (p / SCALE)[..., jnp.array([0, 2])]
    xy = xy / (1.0 + PADDING + 10e-4) + 0.5
    xy = jnp.where(xy >= 1.0, 1.0 - 10e-4, xy)
    xy = jnp.where(xy < 0.0, 0.0, xy)
    xyi = (xy * RESO).astype(jnp.int32)
    idx = xyi[..., 0] + RESO * xyi[..., 1]

    pt = jnp.swapaxes(p, 1, 2)
    net = _knn_conv(pt, params['conv_pos_f'], params['conv_pos_d'])
    net = _fcpos_block0(net, params['fc_pos'], params['block0'])
    for i in range(1, 5):
        pooled = _pool_max(net, idx)
        net = _block(net, pooled, params['block' + str(i)],
                     wcc=params['fc_c'] if i == 4 else None)
    mean_sc = _seg_mean(net, idx)
    return jnp.transpose(mean_sc, (0, 2, 1, 3)).reshape(b, 3 * HID, RESO, RESO)
